# baseline (speedup 1.0000x reference)
import dataclasses
import functools

import jax
import jax.numpy as jnp
from jax import lax
from jax.experimental import pallas as pl
from jax.experimental.pallas import tpu as pltpu
from jax.experimental.pallas import tpu_sc as plsc

N = 10000
E = 320000
D = 128
SW = 16
NC = 2
NS = 16
NL = 16
CHUNK = 128
E_PAD = 327680
EPW = E_PAD // (NC * NS)
CH2 = EPW // CHUNK
MB4 = 128
EPS = E_PAD // NS
CH4M = EPS // (2 * MB4)
N_ACC = 10240
NPS = N_ACC // NS
BE = 1024
BN = 2000


def _mesh():
    return plsc.VectorSubcoreMesh(core_axis_name="c", subcore_axis_name="s",
                                  num_cores=NC, num_subcores=NS)


def _sc_params():
    return pltpu.CompilerParams(use_tc_tiling_on_sc=False)


def _k1_body(x_ref, w_ref, b_ref, h_ref):
    h_ref[...] = (jnp.dot(x_ref[...], w_ref[...],
                          preferred_element_type=jnp.float32) + b_ref[...])


def _k1(x, W_in, b_in):
    return pl.pallas_call(
        _k1_body,
        out_shape=jax.ShapeDtypeStruct((N, D), jnp.float32),
    )(x, W_in, b_in.reshape(1, D))


def _k2(h, pos16, src_p, dst_p):
    @functools.partial(
        pl.kernel,
        out_type=[jax.ShapeDtypeStruct((E_PAD, D), jnp.float32),
                  jax.ShapeDtypeStruct((E_PAD, D), jnp.float32),
                  jax.ShapeDtypeStruct((E_PAD, SW), jnp.float32),
                  jax.ShapeDtypeStruct((E_PAD, SW), jnp.float32)],
        mesh=_mesh(),
        scratch_types=[pltpu.VMEM((CHUNK,), jnp.int32),
                       pltpu.VMEM((CHUNK,), jnp.int32),
                       pltpu.VMEM((CHUNK, D), jnp.float32),
                       pltpu.VMEM((CHUNK, D), jnp.float32),
                       pltpu.VMEM((CHUNK, SW), jnp.float32),
                       pltpu.VMEM((CHUNK, SW), jnp.float32),
                       pltpu.SemaphoreType.DMA],
        compiler_params=_sc_params(),
    )
    def k2(h_hbm, p_hbm, srci_hbm, dsti_hbm, osrc, odst, ops, opd,
           idx_s, idx_d, rows_s, rows_d, prow_s, prow_d, gsem):
        wid = lax.axis_index("s") * NC + lax.axis_index("c")

        @pl.loop(0, CH2)
        def _(c):
            base = wid * EPW + c * CHUNK
            dsl = pl.ds(base, CHUNK)
            pltpu.sync_copy(srci_hbm.at[dsl], idx_s)
            pltpu.sync_copy(dsti_hbm.at[dsl], idx_d)
            g1 = pltpu.async_copy(h_hbm.at[idx_s], rows_s, gsem)
            g2 = pltpu.async_copy(h_hbm.at[idx_d], rows_d, gsem)
            g3 = pltpu.async_copy(p_hbm.at[idx_s], prow_s, gsem)
            g4 = pltpu.async_copy(p_hbm.at[idx_d], prow_d, gsem)
            g1.wait()
            g2.wait()
            g3.wait()
            g4.wait()
            pltpu.sync_copy(rows_s, osrc.at[dsl])
            pltpu.sync_copy(rows_d, odst.at[dsl])
            pltpu.sync_copy(prow_s, ops.at[dsl])
            pltpu.sync_copy(prow_d, opd.at[dsl])

    return k2(h, pos16, src_p, dst_p)


def _k3_body(hs_ref, hd_ref, ps_ref, pd_ref,
             wm1s_ref, wm1d_ref, bm1_ref, wm2_ref, bm2_ref,
             aws_ref, awd_ref, ab_ref,
             we1s_ref, we1d_ref, wd2_ref, be1_ref, we2_ref, be2_ref, wp_ref,
             oa_ref, ob_ref, os_ref):
    i = pl.program_id(0)
    hs = hs_ref[...]
    hd = hd_ref[...]
    f32 = jnp.float32
    pre_m = (jnp.dot(hs, wm1s_ref[...], preferred_element_type=f32)
             + jnp.dot(hd, wm1d_ref[...], preferred_element_type=f32)
             + bm1_ref[...])
    m = (jnp.dot(jnp.maximum(pre_m, 0.0), wm2_ref[...],
                 preferred_element_type=f32) + bm2_ref[...])
    logits = (jnp.dot(hs, aws_ref[...], preferred_element_type=f32)
              + jnp.dot(hd, awd_ref[...], preferred_element_type=f32)
              + ab_ref[...])
    e = jnp.exp(logits)
    rel = ps_ref[:, 0:3] - pd_ref[:, 0:3]
    d2 = jnp.sum(rel * rel, axis=1, keepdims=True)
    pre_e = (jnp.dot(hs, we1s_ref[...], preferred_element_type=f32)
             + jnp.dot(hd, we1d_ref[...], preferred_element_type=f32)
             + d2 * wd2_ref[...]
             + be1_ref[...])
    me = (jnp.dot(jnp.maximum(pre_e, 0.0), we2_ref[...],
                  preferred_element_type=f32) + be2_ref[...])
    coef = jnp.dot(me, wp_ref[...], preferred_element_type=f32)
    eid = i * BE + lax.broadcasted_iota(jnp.int32, (BE, 1), 0)
    mask = jnp.where(eid < E, 1.0, 0.0).astype(f32)
    oa_ref[...] = m * e * mask
    ob_ref[...] = me * mask
    small = jnp.concatenate(
        [e, jnp.ones_like(e), rel * coef, jnp.zeros((BE, SW - 5), f32)], axis=1)
    os_ref[...] = small * mask


def _k3(hsrc, hdst, psrc, pdst, W_m1, b_m1, W_m2, b_m2, a_w, a_b, W_e1, b_e1,
        W_e2, b_e2, W_p):
    grid = (E_PAD // BE,)
    blk = lambda r, c: pl.BlockSpec((r, c), lambda i: (i, 0))
    wblk = lambda r, c: pl.BlockSpec((r, c), lambda i: (0, 0))
    return pl.pallas_call(
        _k3_body,
        grid=grid,
        in_specs=[blk(BE, D), blk(BE, D), blk(BE, SW), blk(BE, SW),
                  wblk(D, D), wblk(D, D), wblk(1, D), wblk(D, D), wblk(1, D),
                  wblk(D, 1), wblk(D, 1), wblk(1, 1),
                  wblk(D, D), wblk(D, D), wblk(1, D), wblk(1, D), wblk(D, D),
                  wblk(1, D), wblk(D, 1)],
        out_specs=[blk(BE, D), blk(BE, D), blk(BE, SW)],
        out_shape=[jax.ShapeDtypeStruct((E_PAD, D), jnp.float32),
                   jax.ShapeDtypeStruct((E_PAD, D), jnp.float32),
                   jax.ShapeDtypeStruct((E_PAD, SW), jnp.float32)],
    )(hsrc, hdst, psrc, pdst,
      W_m1[:D], W_m1[D:], b_m1.reshape(1, D), W_m2, b_m2.reshape(1, D),
      a_w[:D], a_w[D:], a_b.reshape(1, 1),
      W_e1[:D], W_e1[D:2 * D], W_e1[2 * D:2 * D + 1], b_e1.reshape(1, D),
      W_e2, b_e2.reshape(1, D), W_p)


def _k4(oA, oB, oS, dst2d, z128, z16):
    @functools.partial(
        pl.kernel,
        out_type=[jax.ShapeDtypeStruct((N_ACC, D), jnp.float32),
                  jax.ShapeDtypeStruct((N_ACC, D), jnp.float32),
                  jax.ShapeDtypeStruct((N_ACC, SW), jnp.float32)],
        mesh=_mesh(),
        scratch_types=[pltpu.VMEM_SHARED((N_ACC, D), jnp.float32),
                       pltpu.VMEM_SHARED((N_ACC, SW), jnp.float32),
                       pltpu.VMEM((2, MB4 // CHUNK, CHUNK), jnp.int32),
                       pltpu.VMEM((2, MB4, D), jnp.float32),
                       pltpu.VMEM((2, MB4, SW), jnp.float32),
                       pltpu.SemaphoreType.DMA,
                       pltpu.SemaphoreType.DMA],
        compiler_params=_sc_params(),
    )
    def k4(a_hbm, b_hbm, s_hbm, dsti_hbm, z128_hbm, z16_hbm,
           occ, oet, osm, acc, accs, idx_v, val_v, sval_v, lsem0, lsem1):
        cid = lax.axis_index("c")
        sid = lax.axis_index("s")
        rows = pl.ds(sid * NPS, NPS)
        lsems = (lsem0, lsem1)
        pltpu.sync_copy(z128_hbm, acc.at[rows])
        pltpu.sync_copy(z16_hbm, accs.at[rows])
        plsc.subcore_barrier()

        @pl.when(cid == 0)
        def _():
            @pl.loop(0, CH4M)
            def _(m):
                row0 = (sid * EPS + m * (2 * MB4)) // CHUNK
                lcps = []
                for b in range(2):
                    esl = pl.ds((row0 + b * (MB4 // CHUNK)) * CHUNK, MB4)
                    rsl = pl.ds(row0 + b * (MB4 // CHUNK), MB4 // CHUNK)
                    lcps.append((
                        pltpu.async_copy(dsti_hbm.at[rsl], idx_v.at[b],
                                         lsems[b]),
                        pltpu.async_copy(a_hbm.at[esl], val_v.at[b],
                                         lsems[b])))
                for b in range(2):
                    for cp in lcps[b]:
                        cp.wait()
                    for j in range(MB4 // CHUNK):
                        vsl = pl.ds(j * CHUNK, CHUNK)
                        pltpu.sync_copy(val_v.at[b, vsl],
                                        acc.at[idx_v.at[b, j]], add=True)

        @pl.when(cid == 1)
        def _():
            @pl.loop(0, CH4M)
            def _(m):
                row0 = (sid * EPS + m * (2 * MB4)) // CHUNK
                lcps = []
                for b in range(2):
                    esl = pl.ds((row0 + b * (MB4 // CHUNK)) * CHUNK, MB4)
                    rsl = pl.ds(row0 + b * (MB4 // CHUNK), MB4 // CHUNK)
                    lcps.append((
                        pltpu.async_copy(dsti_hbm.at[rsl], idx_v.at[b],
                                         lsems[b]),
                        pltpu.async_copy(b_hbm.at[esl], val_v.at[b],
                                         lsems[b]),
                        pltpu.async_copy(s_hbm.at[esl], sval_v.at[b],
                                         lsems[b])))
                for b in range(2):
                    for cp in lcps[b]:
                        cp.wait()
                    for j in range(MB4 // CHUNK):
                        vsl = pl.ds(j * CHUNK, CHUNK)
                        pltpu.sync_copy(val_v.at[b, vsl],
                                        acc.at[idx_v.at[b, j]], add=True)
                        pltpu.sync_copy(sval_v.at[b, vsl],
                                        accs.at[idx_v.at[b, j]], add=True)

        plsc.subcore_barrier()

        @pl.when(cid == 0)
        def _():
            pltpu.sync_copy(acc.at[rows], occ.at[rows])

        @pl.when(cid == 1)
        def _():
            pltpu.sync_copy(acc.at[rows], oet.at[rows])
            pltpu.sync_copy(accs.at[rows], osm.at[rows])

    return k4(oA, oB, oS, dst2d, z128, z16)


def _k5_body(cca_ref, ccb_ref, ccs_ref, h_ref, pos_ref,
             wf1t_ref, wf1b_ref, bf1_ref, wf2_ref, bf2_ref, g_ref, be_ref,
             out_ref, pu_ref):
    f32 = jnp.float32
    ccs = ccs_ref[...]
    denom = ccs[:, 0:1]
    deg = ccs[:, 1:2]
    pacc = ccs[:, 2:5]
    ccm = cca_ref[...] / (denom + 1e-9)
    et = ccb_ref[...]
    t1 = jnp.maximum(
        jnp.dot(ccm, wf1t_ref[...], preferred_element_type=f32)
        + jnp.dot(et, wf1b_ref[...], preferred_element_type=f32)
        + bf1_ref[...], 0.0)
    s = jnp.dot(t1, wf2_ref[...], preferred_element_type=f32) + bf2_ref[...]
    mx = jnp.max(s, axis=1, keepdims=True)
    es = jnp.exp(s - mx)
    aw = es / jnp.sum(es, axis=1, keepdims=True)
    fused = aw[:, 0:1] * ccm + aw[:, 1:2] * et
    res = fused + h_ref[...]
    mu = jnp.mean(res, axis=1, keepdims=True)
    cen = res - mu
    var = jnp.mean(cen * cen, axis=1, keepdims=True)
    out_ref[...] = cen / jnp.sqrt(var + 1e-5) * g_ref[...] + be_ref[...]
    pu_ref[...] = pos_ref[...] + pacc / (deg + 1.0)


def _k5(ccA, ccB, ccS, h, pos, W_f1, b_f1, W_f2, b_f2, gamma, beta):
    grid = (N // BN,)
    blk = lambda r, c: pl.BlockSpec((r, c), lambda i: (i, 0))
    wblk = lambda r, c: pl.BlockSpec((r, c), lambda i: (0, 0))
    return pl.pallas_call(
        _k5_body,
        grid=grid,
        in_specs=[blk(BN, D), blk(BN, D), blk(BN, SW), blk(BN, D), blk(BN, 3),
                  wblk(D, D), wblk(D, D), wblk(1, D), wblk(D, 2), wblk(1, 2),
                  wblk(1, D), wblk(1, D)],
        out_specs=[blk(BN, D), blk(BN, 3)],
        out_shape=[jax.ShapeDtypeStruct((N, D), jnp.float32),
                   jax.ShapeDtypeStruct((N, 3), jnp.float32)],
    )(ccA, ccB, ccS, h, pos,
      W_f1[:D], W_f1[D:], b_f1.reshape(1, D), W_f2, b_f2.reshape(1, 2),
      gamma.reshape(1, D), beta.reshape(1, D))


def kernel(x, pos, edge_index, W_in, b_in, W_m1, b_m1, W_m2, b_m2, a_w, a_b,
           W_e1, b_e1, W_e2, b_e2, W_p, W_f1, b_f1, W_f2, b_f2, gamma, beta):
    src_p = jnp.pad(edge_index[0], (0, E_PAD - E))
    dst_p = jnp.pad(edge_index[1], (0, E_PAD - E))
    z128 = jnp.zeros((NPS, D), jnp.float32)
    z16 = jnp.zeros((NPS, SW), jnp.float32)
    pos16 = jnp.pad(pos, ((0, 0), (0, SW - 3)))
    h = _k1(x, W_in, b_in)
    hsrc, hdst, psrc, pdst = _k2(h, pos16, src_p, dst_p)
    oA, oB, oS = _k3(hsrc, hdst, psrc, pdst, W_m1, b_m1, W_m2, b_m2, a_w, a_b,
                     W_e1, b_e1, W_e2, b_e2, W_p)
    ccA, ccB, ccS = _k4(oA, oB, oS, dst_p.reshape(E_PAD // CHUNK, CHUNK),
                        z128, z16)
    return _k5(ccA[:N], ccB[:N], ccS[:N], h, pos, W_f1, b_f1, W_f2, b_f2,
               gamma, beta)


# speedup vs baseline: 1.1620x; 1.1620x over previous
import dataclasses
import functools

import jax
import jax.numpy as jnp
from jax import lax
from jax.experimental import pallas as pl
from jax.experimental.pallas import tpu as pltpu
from jax.experimental.pallas import tpu_sc as plsc

N = 10000
E = 320000
D = 128
SW = 16
NC = 2
NS = 16
NL = 16
CHUNK = 128
E_PAD = 323584
EPW = E_PAD // (NC * NS)
CH2 = EPW // CHUNK
EPS = E_PAD // NS
CH4 = EPS // CHUNK
N_ACC = 10240
NPS = N_ACC // NS
BE = 1024
BN = 2000


def _mesh():
    return plsc.VectorSubcoreMesh(core_axis_name="c", subcore_axis_name="s",
                                  num_cores=NC, num_subcores=NS)


def _sc_params():
    return pltpu.CompilerParams(use_tc_tiling_on_sc=False)


def _k1_body(x_ref, w_ref, b_ref, h_ref):
    h_ref[...] = (jnp.dot(x_ref[...], w_ref[...],
                          preferred_element_type=jnp.float32) + b_ref[...])


def _k1(x, W_in, b_in):
    return pl.pallas_call(
        _k1_body,
        out_shape=jax.ShapeDtypeStruct((N, D), jnp.float32),
    )(x, W_in, b_in.reshape(1, D))


def _k2(h, pos16, src_p, dst_p):
    @functools.partial(
        pl.kernel,
        out_type=[jax.ShapeDtypeStruct((E_PAD, D), jnp.float32),
                  jax.ShapeDtypeStruct((E_PAD, D), jnp.float32),
                  jax.ShapeDtypeStruct((E_PAD, SW), jnp.float32),
                  jax.ShapeDtypeStruct((E_PAD, SW), jnp.float32)],
        mesh=_mesh(),
        scratch_types=[pltpu.VMEM((CHUNK,), jnp.int32),
                       pltpu.VMEM((CHUNK,), jnp.int32),
                       pltpu.VMEM((CHUNK, D), jnp.float32),
                       pltpu.VMEM((CHUNK, D), jnp.float32),
                       pltpu.VMEM((CHUNK, SW), jnp.float32),
                       pltpu.VMEM((CHUNK, SW), jnp.float32),
                       pltpu.SemaphoreType.DMA],
        compiler_params=_sc_params(),
    )
    def k2(h_hbm, p_hbm, srci_hbm, dsti_hbm, osrc, odst, ops, opd,
           idx_s, idx_d, rows_s, rows_d, prow_s, prow_d, sem):
        wid = lax.axis_index("s") * NC + lax.axis_index("c")

        @pl.loop(0, CH2)
        def _(c):
            base = wid * EPW + c * CHUNK
            dsl = pl.ds(base, CHUNK)
            pltpu.sync_copy(srci_hbm.at[dsl], idx_s)
            pltpu.sync_copy(dsti_hbm.at[dsl], idx_d)
            cp1 = pltpu.async_copy(h_hbm.at[idx_s], rows_s, sem)
            cp2 = pltpu.async_copy(h_hbm.at[idx_d], rows_d, sem)
            cp3 = pltpu.async_copy(p_hbm.at[idx_s], prow_s, sem)
            cp4 = pltpu.async_copy(p_hbm.at[idx_d], prow_d, sem)
            cp1.wait()
            cp2.wait()
            cp3.wait()
            cp4.wait()
            pltpu.sync_copy(rows_s, osrc.at[dsl])
            pltpu.sync_copy(rows_d, odst.at[dsl])
            pltpu.sync_copy(prow_s, ops.at[dsl])
            pltpu.sync_copy(prow_d, opd.at[dsl])

    return k2(h, pos16, src_p, dst_p)


def _k3_body(hs_ref, hd_ref, ps_ref, pd_ref,
             wm1s_ref, wm1d_ref, bm1_ref, wm2_ref, bm2_ref,
             aws_ref, awd_ref, ab_ref,
             we1s_ref, we1d_ref, wd2_ref, be1_ref, we2_ref, be2_ref, wp_ref,
             oa_ref, ob_ref, os_ref):
    i = pl.program_id(0)
    hs = hs_ref[...]
    hd = hd_ref[...]
    f32 = jnp.float32
    pre_m = (jnp.dot(hs, wm1s_ref[...], preferred_element_type=f32)
             + jnp.dot(hd, wm1d_ref[...], preferred_element_type=f32)
             + bm1_ref[...])
    m = (jnp.dot(jnp.maximum(pre_m, 0.0), wm2_ref[...],
                 preferred_element_type=f32) + bm2_ref[...])
    logits = (jnp.dot(hs, aws_ref[...], preferred_element_type=f32)
              + jnp.dot(hd, awd_ref[...], preferred_element_type=f32)
              + ab_ref[...])
    e = jnp.exp(logits)
    rel = ps_ref[:, 0:3] - pd_ref[:, 0:3]
    d2 = jnp.sum(rel * rel, axis=1, keepdims=True)
    pre_e = (jnp.dot(hs, we1s_ref[...], preferred_element_type=f32)
             + jnp.dot(hd, we1d_ref[...], preferred_element_type=f32)
             + d2 * wd2_ref[...]
             + be1_ref[...])
    me = (jnp.dot(jnp.maximum(pre_e, 0.0), we2_ref[...],
                  preferred_element_type=f32) + be2_ref[...])
    coef = jnp.dot(me, wp_ref[...], preferred_element_type=f32)
    eid = i * BE + lax.broadcasted_iota(jnp.int32, (BE, 1), 0)
    mask = jnp.where(eid < E, 1.0, 0.0).astype(f32)
    oa_ref[...] = m * e * mask
    ob_ref[...] = me * mask
    small = jnp.concatenate(
        [e, jnp.ones_like(e), rel * coef, jnp.zeros((BE, SW - 5), f32)], axis=1)
    os_ref[...] = small * mask


def _k3(hsrc, hdst, psrc, pdst, W_m1, b_m1, W_m2, b_m2, a_w, a_b, W_e1, b_e1,
        W_e2, b_e2, W_p):
    grid = (E_PAD // BE,)
    blk = lambda r, c: pl.BlockSpec((r, c), lambda i: (i, 0))
    wblk = lambda r, c: pl.BlockSpec((r, c), lambda i: (0, 0))
    return pl.pallas_call(
        _k3_body,
        grid=grid,
        in_specs=[blk(BE, D), blk(BE, D), blk(BE, SW), blk(BE, SW),
                  wblk(D, D), wblk(D, D), wblk(1, D), wblk(D, D), wblk(1, D),
                  wblk(D, 1), wblk(D, 1), wblk(1, 1),
                  wblk(D, D), wblk(D, D), wblk(1, D), wblk(1, D), wblk(D, D),
                  wblk(1, D), wblk(D, 1)],
        out_specs=[blk(BE, D), blk(BE, D), blk(BE, SW)],
        out_shape=[jax.ShapeDtypeStruct((E_PAD, D), jnp.float32),
                   jax.ShapeDtypeStruct((E_PAD, D), jnp.float32),
                   jax.ShapeDtypeStruct((E_PAD, SW), jnp.float32)],
    )(hsrc, hdst, psrc, pdst,
      W_m1[:D], W_m1[D:], b_m1.reshape(1, D), W_m2, b_m2.reshape(1, D),
      a_w[:D], a_w[D:], a_b.reshape(1, 1),
      W_e1[:D], W_e1[D:2 * D], W_e1[2 * D:2 * D + 1], b_e1.reshape(1, D),
      W_e2, b_e2.reshape(1, D), W_p)


def _k4(oA, oB, oS, dst_p, z128, z16):
    @functools.partial(
        pl.kernel,
        out_type=[jax.ShapeDtypeStruct((N_ACC, D), jnp.float32),
                  jax.ShapeDtypeStruct((N_ACC, D), jnp.float32),
                  jax.ShapeDtypeStruct((N_ACC, SW), jnp.float32)],
        mesh=_mesh(),
        scratch_types=[pltpu.VMEM_SHARED((N_ACC, D), jnp.float32),
                       pltpu.VMEM_SHARED((N_ACC, SW), jnp.float32),
                       pltpu.VMEM((CHUNK,), jnp.int32),
                       pltpu.VMEM((CHUNK, D), jnp.float32),
                       pltpu.VMEM((CHUNK, SW), jnp.float32)],
        compiler_params=_sc_params(),
    )
    def k4(a_hbm, b_hbm, s_hbm, dsti_hbm, z128_hbm, z16_hbm,
           occ, oet, osm, acc, accs, idx_v, val_v, sval_v):
        cid = lax.axis_index("c")
        sid = lax.axis_index("s")
        rows = pl.ds(sid * NPS, NPS)
        pltpu.sync_copy(z128_hbm, acc.at[rows])
        pltpu.sync_copy(z16_hbm, accs.at[rows])
        plsc.subcore_barrier()

        @pl.when(cid == 0)
        def _():
            @pl.loop(0, CH4)
            def _(c):
                base = sid * EPS + c * CHUNK
                dsl = pl.ds(base, CHUNK)
                pltpu.sync_copy(dsti_hbm.at[dsl], idx_v)
                pltpu.sync_copy(a_hbm.at[dsl], val_v)
                pltpu.sync_copy(s_hbm.at[dsl], sval_v)
                pltpu.sync_copy(val_v, acc.at[idx_v], add=True)
                pltpu.sync_copy(sval_v, accs.at[idx_v], add=True)

        @pl.when(cid == 1)
        def _():
            @pl.loop(0, CH4)
            def _(c):
                base = sid * EPS + c * CHUNK
                dsl = pl.ds(base, CHUNK)
                pltpu.sync_copy(dsti_hbm.at[dsl], idx_v)
                pltpu.sync_copy(b_hbm.at[dsl], val_v)
                pltpu.sync_copy(val_v, acc.at[idx_v], add=True)

        plsc.subcore_barrier()

        @pl.when(cid == 0)
        def _():
            pltpu.sync_copy(acc.at[rows], occ.at[rows])
            pltpu.sync_copy(accs.at[rows], osm.at[rows])

        @pl.when(cid == 1)
        def _():
            pltpu.sync_copy(acc.at[rows], oet.at[rows])

    return k4(oA, oB, oS, dst_p, z128, z16)


def _k5_body(cca_ref, ccb_ref, ccs_ref, h_ref, pos_ref,
             wf1t_ref, wf1b_ref, bf1_ref, wf2_ref, bf2_ref, g_ref, be_ref,
             out_ref, pu_ref):
    f32 = jnp.float32
    ccs = ccs_ref[...]
    denom = ccs[:, 0:1]
    deg = ccs[:, 1:2]
    pacc = ccs[:, 2:5]
    ccm = cca_ref[...] / (denom + 1e-9)
    et = ccb_ref[...]
    t1 = jnp.maximum(
        jnp.dot(ccm, wf1t_ref[...], preferred_element_type=f32)
        + jnp.dot(et, wf1b_ref[...], preferred_element_type=f32)
        + bf1_ref[...], 0.0)
    s = jnp.dot(t1, wf2_ref[...], preferred_element_type=f32) + bf2_ref[...]
    mx = jnp.max(s, axis=1, keepdims=True)
    es = jnp.exp(s - mx)
    aw = es / jnp.sum(es, axis=1, keepdims=True)
    fused = aw[:, 0:1] * ccm + aw[:, 1:2] * et
    res = fused + h_ref[...]
    mu = jnp.mean(res, axis=1, keepdims=True)
    cen = res - mu
    var = jnp.mean(cen * cen, axis=1, keepdims=True)
    out_ref[...] = cen / jnp.sqrt(var + 1e-5) * g_ref[...] + be_ref[...]
    pu_ref[...] = pos_ref[...] + pacc / (deg + 1.0)


def _k5(ccA, ccB, ccS, h, pos, W_f1, b_f1, W_f2, b_f2, gamma, beta):
    grid = (N // BN,)
    blk = lambda r, c: pl.BlockSpec((r, c), lambda i: (i, 0))
    wblk = lambda r, c: pl.BlockSpec((r, c), lambda i: (0, 0))
    return pl.pallas_call(
        _k5_body,
        grid=grid,
        in_specs=[blk(BN, D), blk(BN, D), blk(BN, SW), blk(BN, D), blk(BN, 3),
                  wblk(D, D), wblk(D, D), wblk(1, D), wblk(D, 2), wblk(1, 2),
                  wblk(1, D), wblk(1, D)],
        out_specs=[blk(BN, D), blk(BN, 3)],
        out_shape=[jax.ShapeDtypeStruct((N, D), jnp.float32),
                   jax.ShapeDtypeStruct((N, 3), jnp.float32)],
    )(ccA, ccB, ccS, h, pos,
      W_f1[:D], W_f1[D:], b_f1.reshape(1, D), W_f2, b_f2.reshape(1, 2),
      gamma.reshape(1, D), beta.reshape(1, D))


def kernel(x, pos, edge_index, W_in, b_in, W_m1, b_m1, W_m2, b_m2, a_w, a_b,
           W_e1, b_e1, W_e2, b_e2, W_p, W_f1, b_f1, W_f2, b_f2, gamma, beta):
    src_p = jnp.pad(edge_index[0], (0, E_PAD - E))
    dst_p = jnp.pad(edge_index[1], (0, E_PAD - E))
    z128 = jnp.zeros((NPS, D), jnp.float32)
    z16 = jnp.zeros((NPS, SW), jnp.float32)
    pos16 = jnp.pad(pos, ((0, 0), (0, SW - 3)))
    h = _k1(x, W_in, b_in)
    hsrc, hdst, psrc, pdst = _k2(h, pos16, src_p, dst_p)
    oA, oB, oS = _k3(hsrc, hdst, psrc, pdst, W_m1, b_m1, W_m2, b_m2, a_w, a_b,
                     W_e1, b_e1, W_e2, b_e2, W_p)
    ccA, ccB, ccS = _k4(oA, oB, oS, dst_p, z128, z16)
    return _k5(ccA[:N], ccB[:N], ccS[:N], h, pos, W_f1, b_f1, W_f2, b_f2,
               gamma, beta)


# speedup vs baseline: 1.1654x; 1.0029x over previous
import dataclasses
import functools

import jax
import jax.numpy as jnp
from jax import lax
from jax.experimental import pallas as pl
from jax.experimental.pallas import tpu as pltpu
from jax.experimental.pallas import tpu_sc as plsc

N = 10000
E = 320000
D = 128
SW = 16
NC = 2
NS = 16
NL = 16
CHUNK = 128
E_PAD = 323584
EPW = E_PAD // (NC * NS)
CH2 = EPW // CHUNK
EPS = E_PAD // NS
CH4 = EPS // CHUNK
N_ACC = 10240
NPS = N_ACC // NS
BE = 1024
BN = 2000


def _mesh():
    return plsc.VectorSubcoreMesh(core_axis_name="c", subcore_axis_name="s",
                                  num_cores=NC, num_subcores=NS)


def _sc_params():
    return pltpu.CompilerParams(use_tc_tiling_on_sc=False)


def _k1_body(x_ref, w_ref, b_ref, h_ref):
    h_ref[...] = (jnp.dot(x_ref[...], w_ref[...],
                          preferred_element_type=jnp.float32) + b_ref[...])


def _k1(x, W_in, b_in):
    return pl.pallas_call(
        _k1_body,
        out_shape=jax.ShapeDtypeStruct((N, D), jnp.float32),
    )(x, W_in, b_in.reshape(1, D))


def _k2(h, pos16, src_p, dst_p):
    @functools.partial(
        pl.kernel,
        out_type=[jax.ShapeDtypeStruct((E_PAD, D), jnp.float32),
                  jax.ShapeDtypeStruct((E_PAD, D), jnp.float32),
                  jax.ShapeDtypeStruct((E_PAD, SW), jnp.float32),
                  jax.ShapeDtypeStruct((E_PAD, SW), jnp.float32)],
        mesh=_mesh(),
        scratch_types=[pltpu.VMEM((CHUNK,), jnp.int32),
                       pltpu.VMEM((CHUNK,), jnp.int32),
                       pltpu.VMEM((CHUNK, D), jnp.float32),
                       pltpu.VMEM((CHUNK, D), jnp.float32),
                       pltpu.VMEM((CHUNK, SW), jnp.float32),
                       pltpu.VMEM((CHUNK, SW), jnp.float32),
                       pltpu.SemaphoreType.DMA],
        compiler_params=_sc_params(),
    )
    def k2(h_hbm, p_hbm, srci_hbm, dsti_hbm, osrc, odst, ops, opd,
           idx_s, idx_d, rows_s, rows_d, prow_s, prow_d, sem):
        wid = lax.axis_index("s") * NC + lax.axis_index("c")

        @pl.loop(0, CH2)
        def _(c):
            base = wid * EPW + c * CHUNK
            dsl = pl.ds(base, CHUNK)
            pltpu.sync_copy(srci_hbm.at[dsl], idx_s)
            pltpu.sync_copy(dsti_hbm.at[dsl], idx_d)
            cp1 = pltpu.async_copy(h_hbm.at[idx_s], rows_s, sem)
            cp2 = pltpu.async_copy(h_hbm.at[idx_d], rows_d, sem)
            cp3 = pltpu.async_copy(p_hbm.at[idx_s], prow_s, sem)
            cp4 = pltpu.async_copy(p_hbm.at[idx_d], prow_d, sem)
            cp1.wait()
            cp2.wait()
            cp3.wait()
            cp4.wait()
            pltpu.sync_copy(rows_s, osrc.at[dsl])
            pltpu.sync_copy(rows_d, odst.at[dsl])
            pltpu.sync_copy(prow_s, ops.at[dsl])
            pltpu.sync_copy(prow_d, opd.at[dsl])

    return k2(h, pos16, src_p, dst_p)


def _k3_body(hs_ref, hd_ref, ps_ref, pd_ref,
             wm1s_ref, wm1d_ref, bm1_ref, wm2_ref, bm2_ref,
             aws_ref, awd_ref, ab_ref,
             we1s_ref, we1d_ref, wd2_ref, be1_ref, we2_ref, be2_ref, wp_ref,
             oa_ref, ob_ref, os_ref):
    i = pl.program_id(0)
    hs = hs_ref[...]
    hd = hd_ref[...]
    f32 = jnp.float32
    pre_m = (jnp.dot(hs, wm1s_ref[...], preferred_element_type=f32)
             + jnp.dot(hd, wm1d_ref[...], preferred_element_type=f32)
             + bm1_ref[...])
    m = (jnp.dot(jnp.maximum(pre_m, 0.0), wm2_ref[...],
                 preferred_element_type=f32) + bm2_ref[...])
    logits = (jnp.dot(hs, aws_ref[...], preferred_element_type=f32)
              + jnp.dot(hd, awd_ref[...], preferred_element_type=f32)
              + ab_ref[...])
    e = jnp.exp(logits)
    rel = ps_ref[:, 0:3] - pd_ref[:, 0:3]
    d2 = jnp.sum(rel * rel, axis=1, keepdims=True)
    pre_e = (jnp.dot(hs, we1s_ref[...], preferred_element_type=f32)
             + jnp.dot(hd, we1d_ref[...], preferred_element_type=f32)
             + d2 * wd2_ref[...]
             + be1_ref[...])
    me = (jnp.dot(jnp.maximum(pre_e, 0.0), we2_ref[...],
                  preferred_element_type=f32) + be2_ref[...])
    coef = jnp.dot(me, wp_ref[...], preferred_element_type=f32)
    eid = i * BE + lax.broadcasted_iota(jnp.int32, (BE, 1), 0)
    mask = jnp.where(eid < E, 1.0, 0.0).astype(f32)
    oa_ref[...] = m * e * mask
    ob_ref[...] = me * mask
    small = jnp.concatenate(
        [e, jnp.ones_like(e), rel * coef, jnp.zeros((BE, SW - 5), f32)], axis=1)
    os_ref[...] = small * mask


def _k3(hsrc, hdst, psrc, pdst, W_m1, b_m1, W_m2, b_m2, a_w, a_b, W_e1, b_e1,
        W_e2, b_e2, W_p):
    grid = (E_PAD // BE,)
    blk = lambda r, c: pl.BlockSpec((r, c), lambda i: (i, 0))
    wblk = lambda r, c: pl.BlockSpec((r, c), lambda i: (0, 0))
    return pl.pallas_call(
        _k3_body,
        grid=grid,
        in_specs=[blk(BE, D), blk(BE, D), blk(BE, SW), blk(BE, SW),
                  wblk(D, D), wblk(D, D), wblk(1, D), wblk(D, D), wblk(1, D),
                  wblk(D, 1), wblk(D, 1), wblk(1, 1),
                  wblk(D, D), wblk(D, D), wblk(1, D), wblk(1, D), wblk(D, D),
                  wblk(1, D), wblk(D, 1)],
        out_specs=[blk(BE, D), blk(BE, D), blk(BE, SW)],
        out_shape=[jax.ShapeDtypeStruct((E_PAD, D), jnp.float32),
                   jax.ShapeDtypeStruct((E_PAD, D), jnp.float32),
                   jax.ShapeDtypeStruct((E_PAD, SW), jnp.float32)],
    )(hsrc, hdst, psrc, pdst,
      W_m1[:D], W_m1[D:], b_m1.reshape(1, D), W_m2, b_m2.reshape(1, D),
      a_w[:D], a_w[D:], a_b.reshape(1, 1),
      W_e1[:D], W_e1[D:2 * D], W_e1[2 * D:2 * D + 1], b_e1.reshape(1, D),
      W_e2, b_e2.reshape(1, D), W_p)


def _k4(oA, oB, oS, dst_p, z128, z16):
    @functools.partial(
        pl.kernel,
        out_type=[jax.ShapeDtypeStruct((N_ACC, D), jnp.float32),
                  jax.ShapeDtypeStruct((N_ACC, D), jnp.float32),
                  jax.ShapeDtypeStruct((N_ACC, SW), jnp.float32)],
        mesh=_mesh(),
        scratch_types=[pltpu.VMEM_SHARED((N_ACC, D), jnp.float32),
                       pltpu.VMEM_SHARED((N_ACC, SW), jnp.float32),
                       pltpu.VMEM((CHUNK,), jnp.int32),
                       pltpu.VMEM((CHUNK, D), jnp.float32),
                       pltpu.VMEM((CHUNK, SW), jnp.float32)],
        compiler_params=_sc_params(),
    )
    def k4(a_hbm, b_hbm, s_hbm, dsti_hbm, z128_hbm, z16_hbm,
           occ, oet, osm, acc, accs, idx_v, val_v, sval_v):
        cid = lax.axis_index("c")
        sid = lax.axis_index("s")
        rows = pl.ds(sid * NPS, NPS)
        pltpu.sync_copy(z128_hbm, acc.at[rows])
        pltpu.sync_copy(z16_hbm, accs.at[rows])
        plsc.subcore_barrier()

        @pl.when(cid == 0)
        def _():
            @pl.loop(0, CH4)
            def _(c):
                base = sid * EPS + c * CHUNK
                dsl = pl.ds(base, CHUNK)
                pltpu.sync_copy(dsti_hbm.at[dsl], idx_v)
                pltpu.sync_copy(a_hbm.at[dsl], val_v)
                pltpu.sync_copy(val_v, acc.at[idx_v], add=True)

        @pl.when(cid == 1)
        def _():
            @pl.loop(0, CH4)
            def _(c):
                base = sid * EPS + c * CHUNK
                dsl = pl.ds(base, CHUNK)
                pltpu.sync_copy(dsti_hbm.at[dsl], idx_v)
                pltpu.sync_copy(b_hbm.at[dsl], val_v)
                pltpu.sync_copy(s_hbm.at[dsl], sval_v)
                pltpu.sync_copy(val_v, acc.at[idx_v], add=True)
                pltpu.sync_copy(sval_v, accs.at[idx_v], add=True)

        plsc.subcore_barrier()

        @pl.when(cid == 0)
        def _():
            pltpu.sync_copy(acc.at[rows], occ.at[rows])

        @pl.when(cid == 1)
        def _():
            pltpu.sync_copy(acc.at[rows], oet.at[rows])
            pltpu.sync_copy(accs.at[rows], osm.at[rows])

    return k4(oA, oB, oS, dst_p, z128, z16)


def _k5_body(cca_ref, ccb_ref, ccs_ref, h_ref, pos_ref,
             wf1t_ref, wf1b_ref, bf1_ref, wf2_ref, bf2_ref, g_ref, be_ref,
             out_ref, pu_ref):
    f32 = jnp.float32
    ccs = ccs_ref[...]
    denom = ccs[:, 0:1]
    deg = ccs[:, 1:2]
    pacc = ccs[:, 2:5]
    ccm = cca_ref[...] / (denom + 1e-9)
    et = ccb_ref[...]
    t1 = jnp.maximum(
        jnp.dot(ccm, wf1t_ref[...], preferred_element_type=f32)
        + jnp.dot(et, wf1b_ref[...], preferred_element_type=f32)
        + bf1_ref[...], 0.0)
    s = jnp.dot(t1, wf2_ref[...], preferred_element_type=f32) + bf2_ref[...]
    mx = jnp.max(s, axis=1, keepdims=True)
    es = jnp.exp(s - mx)
    aw = es / jnp.sum(es, axis=1, keepdims=True)
    fused = aw[:, 0:1] * ccm + aw[:, 1:2] * et
    res = fused + h_ref[...]
    mu = jnp.mean(res, axis=1, keepdims=True)
    cen = res - mu
    var = jnp.mean(cen * cen, axis=1, keepdims=True)
    out_ref[...] = cen / jnp.sqrt(var + 1e-5) * g_ref[...] + be_ref[...]
    pu_ref[...] = pos_ref[...] + pacc / (deg + 1.0)


def _k5(ccA, ccB, ccS, h, pos, W_f1, b_f1, W_f2, b_f2, gamma, beta):
    grid = (N // BN,)
    blk = lambda r, c: pl.BlockSpec((r, c), lambda i: (i, 0))
    wblk = lambda r, c: pl.BlockSpec((r, c), lambda i: (0, 0))
    return pl.pallas_call(
        _k5_body,
        grid=grid,
        in_specs=[blk(BN, D), blk(BN, D), blk(BN, SW), blk(BN, D), blk(BN, 3),
                  wblk(D, D), wblk(D, D), wblk(1, D), wblk(D, 2), wblk(1, 2),
                  wblk(1, D), wblk(1, D)],
        out_specs=[blk(BN, D), blk(BN, 3)],
        out_shape=[jax.ShapeDtypeStruct((N, D), jnp.float32),
                   jax.ShapeDtypeStruct((N, 3), jnp.float32)],
    )(ccA, ccB, ccS, h, pos,
      W_f1[:D], W_f1[D:], b_f1.reshape(1, D), W_f2, b_f2.reshape(1, 2),
      gamma.reshape(1, D), beta.reshape(1, D))


def kernel(x, pos, edge_index, W_in, b_in, W_m1, b_m1, W_m2, b_m2, a_w, a_b,
           W_e1, b_e1, W_e2, b_e2, W_p, W_f1, b_f1, W_f2, b_f2, gamma, beta):
    src_p = jnp.pad(edge_index[0], (0, E_PAD - E))
    dst_p = jnp.pad(edge_index[1], (0, E_PAD - E))
    z128 = jnp.zeros((NPS, D), jnp.float32)
    z16 = jnp.zeros((NPS, SW), jnp.float32)
    pos16 = jnp.pad(pos, ((0, 0), (0, SW - 3)))
    h = _k1(x, W_in, b_in)
    hsrc, hdst, psrc, pdst = _k2(h, pos16, src_p, dst_p)
    oA, oB, oS = _k3(hsrc, hdst, psrc, pdst, W_m1, b_m1, W_m2, b_m2, a_w, a_b,
                     W_e1, b_e1, W_e2, b_e2, W_p)
    ccA, ccB, ccS = _k4(oA, oB, oS, dst_p, z128, z16)
    return _k5(ccA[:N], ccB[:N], ccS[:N], h, pos, W_f1, b_f1, W_f2, b_f2,
               gamma, beta)
